# baseline (speedup 1.0000x reference)
import functools

import jax
import jax.numpy as jnp
import numpy as np
from jax import lax
from jax.experimental import pallas as pl
from jax.experimental.pallas import tpu as pltpu
from jax.experimental.pallas import tpu_sc as plsc

NN = 100000
DD = 128
NB = 4096
NT = 10
M1 = NB * (1 + NT)
DW = DD // 2

NC, NS = 2, 16
NW = NC * NS
RPW = M1 // NW
CH = 32
NCHUNK = RPW // CH
NPAIR = NCHUNK // 2
TPAD = 16

_BASE = np.arange(4)[:, None] * 32
_EV = np.arange(16) * 2
_R_IDX = np.concatenate([_BASE + _EV, _BASE + _EV + 1], axis=1).reshape(-1)


def _t1_body(x_ref, ws, bs, wa, ba, wot, wob, sp_ref, ap_ref):
    xb = x_ref[...]
    s = jnp.maximum(jnp.dot(xb, ws[...], preferred_element_type=jnp.float32) + bs[...], 0.0)
    sp_ref[...] = jnp.dot(s, wot[...], preferred_element_type=jnp.float32).astype(jnp.bfloat16)
    a = jnp.maximum(jnp.dot(xb, wa[...], preferred_element_type=jnp.float32) + ba[...], 0.0)
    ap_ref[...] = jnp.dot(a, wob[...], preferred_element_type=jnp.float32).astype(jnp.bfloat16)


_T1_BM = 2000


def _t1(x, ws, bs, wa, ba, wot, wob):
    full = pl.BlockSpec((DD, DD), lambda i: (0, 0))
    row = pl.BlockSpec((1, DD), lambda i: (0, 0))
    blk = pl.BlockSpec((_T1_BM, DD), lambda i: (i, 0))
    return pl.pallas_call(
        _t1_body,
        grid=(NN // _T1_BM,),
        in_specs=[blk, full, row, full, row, full, full],
        out_specs=[blk, blk],
        out_shape=[jax.ShapeDtypeStruct((NN, DD), jnp.bfloat16)] * 2,
    )(x, ws, bs, wa, ba, wot, wob)


def _lo16(w):
    return lax.bitcast_convert_type(jnp.left_shift(w, 16), jnp.float32)


def _hi16(w):
    return lax.bitcast_convert_type(jnp.bitwise_and(w, jnp.int32(-65536)), jnp.float32)


def _sc_body(sp_hbm, ap_hbm, idxs_hbm, idxn_hbm, al_hbm, b_hbm,
             e1t_hbm, e1n_hbm, sq_hbm,
             idxs_v0, idxs_v1, idxn_v0, idxn_v1, al_v0, al_v1,
             self_v0, self_v1, nbr_v0, nbr_v1, out_v0, out_v1, b_v, sq_v,
             sem_s0, sem_n0, sem_s1, sem_n1):
    wid = lax.axis_index("s") * NC + lax.axis_index("c")
    base = wid * RPW
    idxs_v = (idxs_v0, idxs_v1)
    idxn_v = (idxn_v0, idxn_v1)
    al_v = (al_v0, al_v1)
    self_v = (self_v0, self_v1)
    nbr_v = (nbr_v0, nbr_v1)
    out_v = (out_v0, out_v1)
    sem_s = (sem_s0, sem_s1)
    sem_n = (sem_n0, sem_n1)
    pltpu.sync_copy(b_hbm, b_v)
    sq_v[0, :] = jnp.zeros((16,), jnp.float32)
    sq_v[1, :] = jnp.zeros((16,), jnp.float32)

    def issue(buf, r0):
        pltpu.sync_copy(idxs_hbm.at[pl.ds(r0, CH)], idxs_v[buf])
        pltpu.sync_copy(idxn_hbm.at[pl.ds(r0 * NT, CH * NT)], idxn_v[buf])
        pltpu.sync_copy(al_hbm.at[pl.ds(r0, CH)], al_v[buf])
        pltpu.async_copy(sp_hbm.at[idxs_v[buf]], self_v[buf], sem_s[buf])
        pltpu.async_copy(ap_hbm.at[idxn_v[buf]], nbr_v[buf], sem_n[buf])

    def wait(buf):
        pltpu.make_async_copy(sp_hbm.at[idxs_v[buf]], self_v[buf], sem_s[buf]).wait()
        pltpu.make_async_copy(ap_hbm.at[idxn_v[buf]], nbr_v[buf], sem_n[buf]).wait()

    def compute(buf, r0):
        def row(i, sqc):
            al_row = al_v[buf][i, :]
            als = [al_row[t] for t in range(NT)]
            for g in range(4):
                wsl = pl.ds(g * 16, 16)
                ws_ = self_v[buf][i, wsl]
                acc_lo = _lo16(ws_) + b_v[pl.ds(g * 32, 16)]
                acc_hi = _hi16(ws_) + b_v[pl.ds(g * 32 + 16, 16)]
                for t in range(NT):
                    wn = nbr_v[buf][i * NT + t, wsl]
                    acc_lo = acc_lo + als[t] * _lo16(wn)
                    acc_hi = acc_hi + als[t] * _hi16(wn)
                rlo = jnp.maximum(acc_lo, 0.0)
                rhi = jnp.maximum(acc_hi, 0.0)
                out_v[buf][i, pl.ds(g * 32, 16)] = rlo
                out_v[buf][i, pl.ds(g * 32 + 16, 16)] = rhi
                sqc = sqc + rlo * rlo + rhi * rhi
            return sqc

        sqc = lax.fori_loop(0, CH, row, jnp.zeros((16,), jnp.float32))
        is_t = r0 < NB

        @pl.when(is_t)
        def _():
            sq_v[0, :] = sq_v[0, :] + sqc
            pltpu.sync_copy(out_v[buf], e1t_hbm.at[pl.ds(r0, CH)])

        @pl.when(jnp.logical_not(is_t))
        def _():
            sq_v[1, :] = sq_v[1, :] + sqc
            pltpu.sync_copy(out_v[buf], e1n_hbm.at[pl.ds(r0 - NB, CH)])

    issue(0, base)

    def pair(k, carry):
        r_a = base + (2 * k) * CH
        r_b = r_a + CH
        issue(1, r_b)
        wait(0)
        compute(0, r_a)

        @pl.when(k + 1 < NPAIR)
        def _():
            issue(0, r_b + CH)

        wait(1)
        compute(1, r_b)
        return carry

    lax.fori_loop(0, NPAIR, pair, 0)
    pltpu.sync_copy(sq_v, sq_hbm.at[wid])


_sc_call = functools.partial(
    pl.kernel,
    out_type=(
        jax.ShapeDtypeStruct((NB, DD), jnp.float32),
        jax.ShapeDtypeStruct((NB * NT, DD), jnp.float32),
        jax.ShapeDtypeStruct((NW, 2, 16), jnp.float32),
    ),
    mesh=plsc.VectorSubcoreMesh(core_axis_name="c", subcore_axis_name="s"),
    compiler_params=pltpu.CompilerParams(use_tc_tiling_on_sc=False),
    scratch_types=[
        pltpu.VMEM((CH,), jnp.int32),
        pltpu.VMEM((CH,), jnp.int32),
        pltpu.VMEM((CH * NT,), jnp.int32),
        pltpu.VMEM((CH * NT,), jnp.int32),
        pltpu.VMEM((CH, TPAD), jnp.float32),
        pltpu.VMEM((CH, TPAD), jnp.float32),
        pltpu.VMEM((CH, DW), jnp.int32),
        pltpu.VMEM((CH, DW), jnp.int32),
        pltpu.VMEM((CH * NT, DW), jnp.int32),
        pltpu.VMEM((CH * NT, DW), jnp.int32),
        pltpu.VMEM((CH, DD), jnp.float32),
        pltpu.VMEM((CH, DD), jnp.float32),
        pltpu.VMEM((DD,), jnp.float32),
        pltpu.VMEM((2, 16), jnp.float32),
        pltpu.SemaphoreType.DMA,
        pltpu.SemaphoreType.DMA,
        pltpu.SemaphoreType.DMA,
        pltpu.SemaphoreType.DMA,
    ],
)(_sc_body)


_BM0 = 512


def _t3b_body(e1t_ref, e1n_ref, al_ref, ws0, bs0, wa0, ba0, wo0t, wo0b, bo0,
              nus_ref, e0_ref, sq_ref):
    nu1 = nus_ref[0, 0]
    nu2 = nus_ref[0, 1]
    zt = jnp.maximum(
        jnp.dot(e1t_ref[...], ws0[...], preferred_element_type=jnp.float32)
        + nu1 * bs0[...], 0.0)
    al = al_ref[...]
    agg = jnp.zeros((_BM0, DD), jnp.float32)
    for t in range(NT):
        znt = jnp.maximum(
            jnp.dot(e1n_ref[t], wa0[...], preferred_element_type=jnp.float32)
            + nu2 * ba0[...], 0.0)
        agg = agg + al[:, t:t + 1] * znt
    pre = (jnp.dot(zt, wo0t[...], preferred_element_type=jnp.float32) / nu1
           + jnp.dot(agg, wo0b[...], preferred_element_type=jnp.float32) / nu2
           + bo0[...])
    e0 = jnp.maximum(pre, 0.0)
    e0_ref[...] = e0

    @pl.when(pl.program_id(0) == 0)
    def _():
        sq_ref[...] = jnp.zeros_like(sq_ref)

    sq_ref[...] += jnp.sum(e0 * e0, axis=0, keepdims=True)


def _t3b(e1t, e1n3, al1, ws0, bs0, wa0, ba0, wo0t, wo0b, bo0, nus):
    full = pl.BlockSpec((DD, DD), lambda i: (0, 0))
    row = pl.BlockSpec((1, DD), lambda i: (0, 0))
    return pl.pallas_call(
        _t3b_body,
        grid=(NB // _BM0,),
        in_specs=[
            pl.BlockSpec((_BM0, DD), lambda i: (i, 0)),
            pl.BlockSpec((NT, _BM0, DD), lambda i: (0, i, 0)),
            pl.BlockSpec((_BM0, TPAD), lambda i: (i, 0)),
            full, row, full, row, full, full, row,
            pl.BlockSpec((1, 2), lambda i: (0, 0), memory_space=pltpu.SMEM),
        ],
        out_specs=[
            pl.BlockSpec((_BM0, DD), lambda i: (i, 0)),
            pl.BlockSpec((1, DD), lambda i: (0, 0)),
        ],
        out_shape=[
            jax.ShapeDtypeStruct((NB, DD), jnp.float32),
            jax.ShapeDtypeStruct((1, DD), jnp.float32),
        ],
    )(e1t, e1n3, al1, ws0, bs0, wa0, ba0, wo0t, wo0b, bo0, nus)


def _t3c_body(e0_ref, wemb, bemb, nu_ref, out_ref):
    nu0 = nu_ref[0, 0]
    q = jnp.maximum(
        jnp.dot(e0_ref[...], wemb[...], preferred_element_type=jnp.float32)
        + nu0 * bemb[...], 0.0)
    out_ref[...] = q * (1.0 / nu0)


def _t3c(e0, wemb, bemb, nu0):
    return pl.pallas_call(
        _t3c_body,
        grid=(2,),
        in_specs=[
            pl.BlockSpec((NB // 2, DD), lambda i: (i, 0)),
            pl.BlockSpec((DD, DD), lambda i: (0, 0)),
            pl.BlockSpec((1, DD), lambda i: (0, 0)),
            pl.BlockSpec((1, 1), lambda i: (0, 0), memory_space=pltpu.SMEM),
        ],
        out_specs=pl.BlockSpec((NB // 2, DD), lambda i: (i, 0)),
        out_shape=jax.ShapeDtypeStruct((NB, DD), jnp.float32),
    )(e0, wemb, bemb, nu0)


def kernel(x, node_ids, neigh1, alpha1, neigh2, alpha2,
           W_self1, b_self1, W_agg1, b_agg1, W_out1, b_out1,
           W_self0, b_self0, W_agg0, b_agg0, W_out0, b_out0,
           W_emb, b_emb):
    idx_self = jnp.concatenate([node_ids, neigh1.T.reshape(-1)])
    nbr_n = neigh2.reshape(NB, NT, NT).transpose(1, 0, 2).reshape(NB * NT, NT)
    idx_nbr = jnp.concatenate([neigh1, nbr_n], axis=0).reshape(-1)
    al_n = alpha2.reshape(NB, NT, NT).transpose(1, 0, 2).reshape(NB * NT, NT)
    alpha_cat = jnp.pad(jnp.concatenate([alpha1, al_n], axis=0),
                        ((0, 0), (0, TPAD - NT)))
    alpha1_pad = alpha_cat[:NB]
    rperm = jnp.asarray(_R_IDX)

    Sp, Ap = _t1(x, W_self1, b_self1.reshape(1, DD), W_agg1,
                 b_agg1.reshape(1, DD), W_out1[:DD], W_out1[DD:])
    Spw = lax.bitcast_convert_type(Sp.reshape(NN, DW, 2), jnp.int32)
    Apw = lax.bitcast_convert_type(Ap.reshape(NN, DW, 2), jnp.int32)

    e1_t, e1_n, sqp = _sc_call(Spw, Apw, idx_self, idx_nbr, alpha_cat,
                               b_out1[rperm])
    nu1 = jnp.sqrt(jnp.sum(sqp[:, 0, :]))
    nu2 = jnp.sqrt(jnp.sum(sqp[:, 1, :]))

    e1n3 = e1_n.reshape(NT, NB, DD)
    nus = jnp.stack([nu1, nu2]).reshape(1, 2)
    e0, sq0 = _t3b(e1_t, e1n3, alpha1_pad, W_self0[rperm],
                   b_self0.reshape(1, DD), W_agg0[rperm],
                   b_agg0.reshape(1, DD), W_out0[:DD], W_out0[DD:],
                   b_out0.reshape(1, DD), nus)
    nu0 = jnp.sqrt(jnp.sum(sq0)).reshape(1, 1)

    return _t3c(e0, W_emb, b_emb.reshape(1, DD), nu0)


# speedup vs baseline: 2.5519x; 2.5519x over previous
import functools

import jax
import jax.numpy as jnp
import numpy as np
from jax import lax
from jax.experimental import pallas as pl
from jax.experimental.pallas import tpu as pltpu
from jax.experimental.pallas import tpu_sc as plsc

NN = 100000
DD = 128
NB = 4096
NT = 10
M1 = NB * (1 + NT)
DW = DD // 2

NC, NS = 2, 16
NW = NC * NS
RPW = M1 // NW
CH = 32
NCHUNK = RPW // CH
NPAIR = NCHUNK // 2
TPAD = 16

_BASE = np.arange(4)[:, None] * 32
_EV = np.arange(16) * 2
_R_IDX = np.concatenate([_BASE + _EV, _BASE + _EV + 1], axis=1).reshape(-1)


def _t1_body(x_ref, ws, bs, wa, ba, wot, wob, sp_ref, ap_ref):
    xb = x_ref[...]
    s = jnp.maximum(jnp.dot(xb, ws[...], preferred_element_type=jnp.float32) + bs[...], 0.0)
    sp_ref[...] = jnp.dot(s, wot[...], preferred_element_type=jnp.float32)
    a = jnp.maximum(jnp.dot(xb, wa[...], preferred_element_type=jnp.float32) + ba[...], 0.0)
    ap_ref[...] = jnp.dot(a, wob[...], preferred_element_type=jnp.float32)


_T1_BM = 2000


def _t1(x, ws, bs, wa, ba, wot, wob):
    full = pl.BlockSpec((DD, DD), lambda i: (0, 0))
    row = pl.BlockSpec((1, DD), lambda i: (0, 0))
    blk = pl.BlockSpec((_T1_BM, DD), lambda i: (i, 0))
    return pl.pallas_call(
        _t1_body,
        grid=(NN // _T1_BM,),
        in_specs=[blk, full, row, full, row, full, full],
        out_specs=[blk, blk],
        out_shape=[jax.ShapeDtypeStruct((NN, DD), jnp.float32)] * 2,
    )(x, ws, bs, wa, ba, wot, wob)


def _lo16(w):
    return lax.bitcast_convert_type(jnp.left_shift(w, 16), jnp.float32)


def _hi16(w):
    return lax.bitcast_convert_type(jnp.bitwise_and(w, jnp.int32(-65536)), jnp.float32)


def _sc_body(sp_hbm, ap_hbm, idxs_hbm, idxn_hbm, al_hbm, b_hbm,
             e1t_hbm, e1n_hbm, sq_hbm,
             idxs_v0, idxs_v1, idxn_v0, idxn_v1, al_v0, al_v1,
             self_v0, self_v1, nbr_v0, nbr_v1, out_v0, out_v1, b_v, sq_v,
             sem_s0, sem_n0, sem_s1, sem_n1):
    wid = lax.axis_index("s") * NC + lax.axis_index("c")
    base = wid * RPW
    idxs_v = (idxs_v0, idxs_v1)
    idxn_v = (idxn_v0, idxn_v1)
    al_v = (al_v0, al_v1)
    self_v = (self_v0, self_v1)
    nbr_v = (nbr_v0, nbr_v1)
    out_v = (out_v0, out_v1)
    sem_s = (sem_s0, sem_s1)
    sem_n = (sem_n0, sem_n1)
    pltpu.sync_copy(b_hbm, b_v)
    sq_v[0, :] = jnp.zeros((16,), jnp.float32)
    sq_v[1, :] = jnp.zeros((16,), jnp.float32)

    def issue(buf, r0):
        pltpu.sync_copy(idxs_hbm.at[pl.ds(r0, CH)], idxs_v[buf])
        pltpu.sync_copy(idxn_hbm.at[pl.ds(r0 * NT, CH * NT)], idxn_v[buf])
        pltpu.sync_copy(al_hbm.at[pl.ds(r0, CH)], al_v[buf])
        pltpu.async_copy(sp_hbm.at[idxs_v[buf]], self_v[buf], sem_s[buf])
        pltpu.async_copy(ap_hbm.at[idxn_v[buf]], nbr_v[buf], sem_n[buf])

    def wait(buf):
        pltpu.make_async_copy(sp_hbm.at[idxs_v[buf]], self_v[buf], sem_s[buf]).wait()
        pltpu.make_async_copy(ap_hbm.at[idxn_v[buf]], nbr_v[buf], sem_n[buf]).wait()

    def compute(buf, r0):
        def row(i, sqc):
            al_row = al_v[buf][i, :]
            als = [al_row[t] for t in range(NT)]
            for c in range(DD // 16):
                sl = pl.ds(c * 16, 16)
                acc = self_v[buf][i, sl] + b_v[sl]
                for t in range(NT):
                    acc = acc + als[t] * nbr_v[buf][i * NT + t, sl]
                r = jnp.maximum(acc, 0.0)
                out_v[buf][i, sl] = r
                sqc = sqc + r * r
            return sqc

        sqc = lax.fori_loop(0, CH, row, jnp.zeros((16,), jnp.float32))
        is_t = r0 < NB

        @pl.when(is_t)
        def _():
            sq_v[0, :] = sq_v[0, :] + sqc
            pltpu.sync_copy(out_v[buf], e1t_hbm.at[pl.ds(r0, CH)])

        @pl.when(jnp.logical_not(is_t))
        def _():
            sq_v[1, :] = sq_v[1, :] + sqc
            pltpu.sync_copy(out_v[buf], e1n_hbm.at[pl.ds(r0 - NB, CH)])

    issue(0, base)

    def pair(k, carry):
        r_a = base + (2 * k) * CH
        r_b = r_a + CH
        issue(1, r_b)
        wait(0)
        compute(0, r_a)

        @pl.when(k + 1 < NPAIR)
        def _():
            issue(0, r_b + CH)

        wait(1)
        compute(1, r_b)
        return carry

    lax.fori_loop(0, NPAIR, pair, 0)
    pltpu.sync_copy(sq_v, sq_hbm.at[wid])


_sc_call = functools.partial(
    pl.kernel,
    out_type=(
        jax.ShapeDtypeStruct((NB, DD), jnp.float32),
        jax.ShapeDtypeStruct((NB * NT, DD), jnp.float32),
        jax.ShapeDtypeStruct((NW, 2, 16), jnp.float32),
    ),
    mesh=plsc.VectorSubcoreMesh(core_axis_name="c", subcore_axis_name="s"),
    scratch_types=[
        pltpu.VMEM((CH,), jnp.int32),
        pltpu.VMEM((CH,), jnp.int32),
        pltpu.VMEM((CH * NT,), jnp.int32),
        pltpu.VMEM((CH * NT,), jnp.int32),
        pltpu.VMEM((CH, TPAD), jnp.float32),
        pltpu.VMEM((CH, TPAD), jnp.float32),
        pltpu.VMEM((CH, DD), jnp.float32),
        pltpu.VMEM((CH, DD), jnp.float32),
        pltpu.VMEM((CH * NT, DD), jnp.float32),
        pltpu.VMEM((CH * NT, DD), jnp.float32),
        pltpu.VMEM((CH, DD), jnp.float32),
        pltpu.VMEM((CH, DD), jnp.float32),
        pltpu.VMEM((DD,), jnp.float32),
        pltpu.VMEM((2, 16), jnp.float32),
        pltpu.SemaphoreType.DMA,
        pltpu.SemaphoreType.DMA,
        pltpu.SemaphoreType.DMA,
        pltpu.SemaphoreType.DMA,
    ],
)(_sc_body)


_BM0 = 512


def _t3b_body(e1t_ref, e1n_ref, al_ref, ws0, bs0, wa0, ba0, wo0t, wo0b, bo0,
              nus_ref, e0_ref, sq_ref):
    nu1 = nus_ref[0, 0]
    nu2 = nus_ref[0, 1]
    zt = jnp.maximum(
        jnp.dot(e1t_ref[...], ws0[...], preferred_element_type=jnp.float32)
        + nu1 * bs0[...], 0.0)
    al = al_ref[...]
    agg = jnp.zeros((_BM0, DD), jnp.float32)
    for t in range(NT):
        znt = jnp.maximum(
            jnp.dot(e1n_ref[t], wa0[...], preferred_element_type=jnp.float32)
            + nu2 * ba0[...], 0.0)
        agg = agg + al[:, t:t + 1] * znt
    pre = (jnp.dot(zt, wo0t[...], preferred_element_type=jnp.float32) / nu1
           + jnp.dot(agg, wo0b[...], preferred_element_type=jnp.float32) / nu2
           + bo0[...])
    e0 = jnp.maximum(pre, 0.0)
    e0_ref[...] = e0

    @pl.when(pl.program_id(0) == 0)
    def _():
        sq_ref[...] = jnp.zeros_like(sq_ref)

    sq_ref[...] += jnp.sum(e0 * e0, axis=0, keepdims=True)


def _t3b(e1t, e1n3, al1, ws0, bs0, wa0, ba0, wo0t, wo0b, bo0, nus):
    full = pl.BlockSpec((DD, DD), lambda i: (0, 0))
    row = pl.BlockSpec((1, DD), lambda i: (0, 0))
    return pl.pallas_call(
        _t3b_body,
        grid=(NB // _BM0,),
        in_specs=[
            pl.BlockSpec((_BM0, DD), lambda i: (i, 0)),
            pl.BlockSpec((NT, _BM0, DD), lambda i: (0, i, 0)),
            pl.BlockSpec((_BM0, TPAD), lambda i: (i, 0)),
            full, row, full, row, full, full, row,
            pl.BlockSpec((1, 2), lambda i: (0, 0), memory_space=pltpu.SMEM),
        ],
        out_specs=[
            pl.BlockSpec((_BM0, DD), lambda i: (i, 0)),
            pl.BlockSpec((1, DD), lambda i: (0, 0)),
        ],
        out_shape=[
            jax.ShapeDtypeStruct((NB, DD), jnp.float32),
            jax.ShapeDtypeStruct((1, DD), jnp.float32),
        ],
    )(e1t, e1n3, al1, ws0, bs0, wa0, ba0, wo0t, wo0b, bo0, nus)


def _t3c_body(e0_ref, wemb, bemb, nu_ref, out_ref):
    nu0 = nu_ref[0, 0]
    q = jnp.maximum(
        jnp.dot(e0_ref[...], wemb[...], preferred_element_type=jnp.float32)
        + nu0 * bemb[...], 0.0)
    out_ref[...] = q * (1.0 / nu0)


def _t3c(e0, wemb, bemb, nu0):
    return pl.pallas_call(
        _t3c_body,
        grid=(2,),
        in_specs=[
            pl.BlockSpec((NB // 2, DD), lambda i: (i, 0)),
            pl.BlockSpec((DD, DD), lambda i: (0, 0)),
            pl.BlockSpec((1, DD), lambda i: (0, 0)),
            pl.BlockSpec((1, 1), lambda i: (0, 0), memory_space=pltpu.SMEM),
        ],
        out_specs=pl.BlockSpec((NB // 2, DD), lambda i: (i, 0)),
        out_shape=jax.ShapeDtypeStruct((NB, DD), jnp.float32),
    )(e0, wemb, bemb, nu0)


def kernel(x, node_ids, neigh1, alpha1, neigh2, alpha2,
           W_self1, b_self1, W_agg1, b_agg1, W_out1, b_out1,
           W_self0, b_self0, W_agg0, b_agg0, W_out0, b_out0,
           W_emb, b_emb):
    idx_self = jnp.concatenate([node_ids, neigh1.T.reshape(-1)])
    nbr_n = neigh2.reshape(NB, NT, NT).transpose(1, 0, 2).reshape(NB * NT, NT)
    idx_nbr = jnp.concatenate([neigh1, nbr_n], axis=0).reshape(-1)
    al_n = alpha2.reshape(NB, NT, NT).transpose(1, 0, 2).reshape(NB * NT, NT)
    alpha_cat = jnp.pad(jnp.concatenate([alpha1, al_n], axis=0),
                        ((0, 0), (0, TPAD - NT)))
    alpha1_pad = alpha_cat[:NB]
    rperm = jnp.asarray(_R_IDX)

    Sp, Ap = _t1(x, W_self1, b_self1.reshape(1, DD), W_agg1,
                 b_agg1.reshape(1, DD), W_out1[:DD], W_out1[DD:])

    e1_t, e1_n, sqp = _sc_call(Sp, Ap, idx_self, idx_nbr, alpha_cat, b_out1)
    nu1 = jnp.sqrt(jnp.sum(sqp[:, 0, :]))
    nu2 = jnp.sqrt(jnp.sum(sqp[:, 1, :]))

    e1n3 = e1_n.reshape(NT, NB, DD)
    nus = jnp.stack([nu1, nu2]).reshape(1, 2)
    e0, sq0 = _t3b(e1_t, e1n3, alpha1_pad, W_self0,
                   b_self0.reshape(1, DD), W_agg0,
                   b_agg0.reshape(1, DD), W_out0[:DD], W_out0[DD:],
                   b_out0.reshape(1, DD), nus)
    nu0 = jnp.sqrt(jnp.sum(sq0)).reshape(1, 1)

    return _t3c(e0, W_emb, b_emb.reshape(1, DD), nu0)


# speedup vs baseline: 2.8804x; 1.1287x over previous
import functools

import jax
import jax.numpy as jnp
from jax import lax
from jax.experimental import pallas as pl
from jax.experimental.pallas import tpu as pltpu
from jax.experimental.pallas import tpu_sc as plsc

NN = 100000
DD = 128
NB = 4096
NT = 10
M1 = NB * (1 + NT)

NC, NS = 2, 16
NW = NC * NS
RPW = M1 // NW
CH = 32
NCHUNK = RPW // CH
NPAIR = NCHUNK // 2


def _t1_body(x_ref, ws, bs, wa, ba, wot, wob, sp_ref, ap_ref):
    xb = x_ref[...]
    s = jnp.maximum(jnp.dot(xb, ws[...], preferred_element_type=jnp.float32) + bs[...], 0.0)
    sp_ref[...] = jnp.dot(s, wot[...], preferred_element_type=jnp.float32)
    a = jnp.maximum(jnp.dot(xb, wa[...], preferred_element_type=jnp.float32) + ba[...], 0.0)
    ap_ref[...] = jnp.dot(a, wob[...], preferred_element_type=jnp.float32)


_T1_BM = 2000


def _t1(x, ws, bs, wa, ba, wot, wob):
    full = pl.BlockSpec((DD, DD), lambda i: (0, 0))
    row = pl.BlockSpec((1, DD), lambda i: (0, 0))
    blk = pl.BlockSpec((_T1_BM, DD), lambda i: (i, 0))
    return pl.pallas_call(
        _t1_body,
        grid=(NN // _T1_BM,),
        in_specs=[blk, full, row, full, row, full, full],
        out_specs=[blk, blk],
        out_shape=[jax.ShapeDtypeStruct((NN, DD), jnp.float32)] * 2,
    )(x, ws, bs, wa, ba, wot, wob)


def _sc_body(sp_hbm, ap_hbm, nid_hbm, n1f_hbm, n2f_hbm, a1f_hbm, a2f_hbm,
             b_hbm, e1t_hbm, e1n_hbm, sq_hbm,
             idxs_v0, idxs_v1, idxn_v0, idxn_v1, al_v0, al_v1,
             self_v0, self_v1, nbr_v0, nbr_v1, out_v0, out_v1, b_v, sq_v,
             sem_s0, sem_n0, sem_s1, sem_n1):
    wid = lax.axis_index("s") * NC + lax.axis_index("c")
    base = wid * RPW
    idxs_v = (idxs_v0, idxs_v1)
    idxn_v = (idxn_v0, idxn_v1)
    al_v = (al_v0, al_v1)
    self_v = (self_v0, self_v1)
    nbr_v = (nbr_v0, nbr_v1)
    out_v = (out_v0, out_v1)
    sem_s = (sem_s0, sem_s1)
    sem_n = (sem_n0, sem_n1)
    pltpu.sync_copy(b_hbm, b_v)
    sq_v[0, :] = jnp.zeros((16,), jnp.float32)
    sq_v[1, :] = jnp.zeros((16,), jnp.float32)

    def issue(buf, r0):
        @pl.when(r0 < NB)
        def _():
            pltpu.sync_copy(nid_hbm.at[pl.ds(r0, CH)], idxs_v[buf])
            pltpu.sync_copy(n1f_hbm.at[pl.ds(r0 * NT, CH * NT)], idxn_v[buf])
            pltpu.sync_copy(a1f_hbm.at[pl.ds(r0 * NT, CH * NT)],
                            al_v[buf].at[pl.ds(0, CH * NT)])

        @pl.when(r0 >= NB)
        def _():
            e0 = r0 - NB
            pltpu.sync_copy(n1f_hbm.at[pl.ds(e0, CH)], idxs_v[buf])
            pltpu.sync_copy(n2f_hbm.at[pl.ds(e0 * NT, CH * NT)], idxn_v[buf])
            pltpu.sync_copy(a2f_hbm.at[pl.ds(e0 * NT, CH * NT)],
                            al_v[buf].at[pl.ds(0, CH * NT)])

        pltpu.async_copy(sp_hbm.at[idxs_v[buf]], self_v[buf], sem_s[buf])
        pltpu.async_copy(ap_hbm.at[idxn_v[buf]], nbr_v[buf], sem_n[buf])

    def wait(buf):
        pltpu.make_async_copy(sp_hbm.at[idxs_v[buf]], self_v[buf], sem_s[buf]).wait()
        pltpu.make_async_copy(ap_hbm.at[idxn_v[buf]], nbr_v[buf], sem_n[buf]).wait()

    def compute(buf, r0):
        def row(i, sqc):
            al_vec = al_v[buf][pl.ds(i * NT, 16)]
            als = [al_vec[t] for t in range(NT)]
            for c in range(DD // 16):
                sl = pl.ds(c * 16, 16)
                acc = self_v[buf][i, sl] + b_v[sl]
                for t in range(NT):
                    acc = acc + als[t] * nbr_v[buf][i * NT + t, sl]
                r = jnp.maximum(acc, 0.0)
                out_v[buf][i, sl] = r
                sqc = sqc + r * r
            return sqc

        sqc = lax.fori_loop(0, CH, row, jnp.zeros((16,), jnp.float32))

        @pl.when(r0 < NB)
        def _():
            sq_v[0, :] = sq_v[0, :] + sqc
            pltpu.sync_copy(out_v[buf], e1t_hbm.at[pl.ds(r0, CH)])

        @pl.when(r0 >= NB)
        def _():
            sq_v[1, :] = sq_v[1, :] + sqc
            pltpu.sync_copy(out_v[buf], e1n_hbm.at[pl.ds(r0 - NB, CH)])

    issue(0, base)

    def pair(k, carry):
        r_a = base + (2 * k) * CH
        r_b = r_a + CH
        issue(1, r_b)
        wait(0)
        compute(0, r_a)

        @pl.when(k + 1 < NPAIR)
        def _():
            issue(0, r_b + CH)

        wait(1)
        compute(1, r_b)
        return carry

    lax.fori_loop(0, NPAIR, pair, 0)
    pltpu.sync_copy(sq_v, sq_hbm.at[wid])


_sc_call = functools.partial(
    pl.kernel,
    out_type=(
        jax.ShapeDtypeStruct((NB, DD), jnp.float32),
        jax.ShapeDtypeStruct((NB * NT, DD), jnp.float32),
        jax.ShapeDtypeStruct((NW, 2, 16), jnp.float32),
    ),
    mesh=plsc.VectorSubcoreMesh(core_axis_name="c", subcore_axis_name="s"),
    scratch_types=[
        pltpu.VMEM((CH,), jnp.int32),
        pltpu.VMEM((CH,), jnp.int32),
        pltpu.VMEM((CH * NT,), jnp.int32),
        pltpu.VMEM((CH * NT,), jnp.int32),
        pltpu.VMEM((CH * NT + 16,), jnp.float32),
        pltpu.VMEM((CH * NT + 16,), jnp.float32),
        pltpu.VMEM((CH, DD), jnp.float32),
        pltpu.VMEM((CH, DD), jnp.float32),
        pltpu.VMEM((CH * NT, DD), jnp.float32),
        pltpu.VMEM((CH * NT, DD), jnp.float32),
        pltpu.VMEM((CH, DD), jnp.float32),
        pltpu.VMEM((CH, DD), jnp.float32),
        pltpu.VMEM((DD,), jnp.float32),
        pltpu.VMEM((2, 16), jnp.float32),
        pltpu.SemaphoreType.DMA,
        pltpu.SemaphoreType.DMA,
        pltpu.SemaphoreType.DMA,
        pltpu.SemaphoreType.DMA,
    ],
)(_sc_body)


_BM0 = 512


def _t3b_body(e1t_ref, e1n_ref, al_ref, ws0, bs0, wa0, ba0, wo0t, wo0b, bo0,
              nus_ref, e0_ref, sq_ref):
    nu1 = nus_ref[0, 0]
    nu2 = nus_ref[0, 1]
    zt = jnp.maximum(
        jnp.dot(e1t_ref[...], ws0[...], preferred_element_type=jnp.float32)
        + nu1 * bs0[...], 0.0)
    al = al_ref[...]
    agg = jnp.zeros((_BM0, DD), jnp.float32)
    for t in range(NT):
        znt = jnp.maximum(
            jnp.dot(e1n_ref[:, t, :], wa0[...], preferred_element_type=jnp.float32)
            + nu2 * ba0[...], 0.0)
        agg = agg + al[:, t:t + 1] * znt
    pre = (jnp.dot(zt, wo0t[...], preferred_element_type=jnp.float32) / nu1
           + jnp.dot(agg, wo0b[...], preferred_element_type=jnp.float32) / nu2
           + bo0[...])
    e0 = jnp.maximum(pre, 0.0)
    e0_ref[...] = e0

    @pl.when(pl.program_id(0) == 0)
    def _():
        sq_ref[...] = jnp.zeros_like(sq_ref)

    sq_ref[...] += jnp.sum(e0 * e0, axis=0, keepdims=True)


def _t3b(e1t, e1n3, al1, ws0, bs0, wa0, ba0, wo0t, wo0b, bo0, nus):
    full = pl.BlockSpec((DD, DD), lambda i: (0, 0))
    row = pl.BlockSpec((1, DD), lambda i: (0, 0))
    return pl.pallas_call(
        _t3b_body,
        grid=(NB // _BM0,),
        in_specs=[
            pl.BlockSpec((_BM0, DD), lambda i: (i, 0)),
            pl.BlockSpec((_BM0, NT, DD), lambda i: (i, 0, 0)),
            pl.BlockSpec((_BM0, NT), lambda i: (i, 0)),
            full, row, full, row, full, full, row,
            pl.BlockSpec((1, 2), lambda i: (0, 0), memory_space=pltpu.SMEM),
        ],
        out_specs=[
            pl.BlockSpec((_BM0, DD), lambda i: (i, 0)),
            pl.BlockSpec((1, DD), lambda i: (0, 0)),
        ],
        out_shape=[
            jax.ShapeDtypeStruct((NB, DD), jnp.float32),
            jax.ShapeDtypeStruct((1, DD), jnp.float32),
        ],
    )(e1t, e1n3, al1, ws0, bs0, wa0, ba0, wo0t, wo0b, bo0, nus)


def _t3c_body(e0_ref, wemb, bemb, nu_ref, out_ref):
    nu0 = nu_ref[0, 0]
    q = jnp.maximum(
        jnp.dot(e0_ref[...], wemb[...], preferred_element_type=jnp.float32)
        + nu0 * bemb[...], 0.0)
    out_ref[...] = q * (1.0 / nu0)


def _t3c(e0, wemb, bemb, nu0):
    return pl.pallas_call(
        _t3c_body,
        grid=(2,),
        in_specs=[
            pl.BlockSpec((NB // 2, DD), lambda i: (i, 0)),
            pl.BlockSpec((DD, DD), lambda i: (0, 0)),
            pl.BlockSpec((1, DD), lambda i: (0, 0)),
            pl.BlockSpec((1, 1), lambda i: (0, 0), memory_space=pltpu.SMEM),
        ],
        out_specs=pl.BlockSpec((NB // 2, DD), lambda i: (i, 0)),
        out_shape=jax.ShapeDtypeStruct((NB, DD), jnp.float32),
    )(e0, wemb, bemb, nu0)


def kernel(x, node_ids, neigh1, alpha1, neigh2, alpha2,
           W_self1, b_self1, W_agg1, b_agg1, W_out1, b_out1,
           W_self0, b_self0, W_agg0, b_agg0, W_out0, b_out0,
           W_emb, b_emb):
    Sp, Ap = _t1(x, W_self1, b_self1.reshape(1, DD), W_agg1,
                 b_agg1.reshape(1, DD), W_out1[:DD], W_out1[DD:])

    e1_t, e1_n, sqp = _sc_call(Sp, Ap, node_ids, neigh1.reshape(-1),
                               neigh2.reshape(-1), alpha1.reshape(-1),
                               alpha2.reshape(-1), b_out1)
    nu1 = jnp.sqrt(jnp.sum(sqp[:, 0, :]))
    nu2 = jnp.sqrt(jnp.sum(sqp[:, 1, :]))

    e1n3 = e1_n.reshape(NB, NT, DD)
    nus = jnp.stack([nu1, nu2]).reshape(1, 2)
    e0, sq0 = _t3b(e1_t, e1n3, alpha1, W_self0, b_self0.reshape(1, DD),
                   W_agg0, b_agg0.reshape(1, DD), W_out0[:DD], W_out0[DD:],
                   b_out0.reshape(1, DD), nus)
    nu0 = jnp.sqrt(jnp.sum(sq0)).reshape(1, 1)

    return _t3c(e0, W_emb, b_emb.reshape(1, DD), nu0)


# speedup vs baseline: 2.9543x; 1.0257x over previous
import functools

import jax
import jax.numpy as jnp
from jax import lax
from jax.experimental import pallas as pl
from jax.experimental.pallas import tpu as pltpu
from jax.experimental.pallas import tpu_sc as plsc

NN = 100000
DD = 128
NB = 4096
NT = 10
M1 = NB * (1 + NT)

NC, NS = 2, 16
NW = NC * NS
RPW = M1 // NW
CH = 32
NCHUNK = RPW // CH
NPAIR = NCHUNK // 2


def _t1_body(x_ref, ws, bs, wa, ba, wot, wob, sp_ref, ap_ref):
    xb = x_ref[...]
    s = jnp.maximum(jnp.dot(xb, ws[...], preferred_element_type=jnp.float32) + bs[...], 0.0)
    sp_ref[...] = jnp.dot(s, wot[...], preferred_element_type=jnp.float32)
    a = jnp.maximum(jnp.dot(xb, wa[...], preferred_element_type=jnp.float32) + ba[...], 0.0)
    ap_ref[...] = jnp.dot(a, wob[...], preferred_element_type=jnp.float32)


_T1_BM = 2000


def _t1(x, ws, bs, wa, ba, wot, wob):
    full = pl.BlockSpec((DD, DD), lambda i: (0, 0))
    row = pl.BlockSpec((1, DD), lambda i: (0, 0))
    blk = pl.BlockSpec((_T1_BM, DD), lambda i: (i, 0))
    return pl.pallas_call(
        _t1_body,
        grid=(NN // _T1_BM,),
        in_specs=[blk, full, row, full, row, full, full],
        out_specs=[blk, blk],
        out_shape=[jax.ShapeDtypeStruct((NN, DD), jnp.float32)] * 2,
    )(x, ws, bs, wa, ba, wot, wob)


def _sc_body(sp_hbm, ap_hbm, nid_hbm, n1f_hbm, n2f_hbm, a1f_hbm, a2f_hbm,
             b_hbm, e1t_hbm, e1n_hbm, sqt_hbm, sqn_hbm,
             idxs_v0, idxs_v1, idxn_v0, idxn_v1, al_v0, al_v1,
             self_v0, self_v1, nbr_v0, nbr_v1, out_v0, out_v1, b_v, sq_v,
             sem_s0, sem_n0, sem_s1, sem_n1, sem_w0, sem_w1):
    wid = lax.axis_index("s") * NC + lax.axis_index("c")
    base = wid * RPW
    idxs_v = (idxs_v0, idxs_v1)
    idxn_v = (idxn_v0, idxn_v1)
    al_v = (al_v0, al_v1)
    self_v = (self_v0, self_v1)
    nbr_v = (nbr_v0, nbr_v1)
    out_v = (out_v0, out_v1)
    sem_s = (sem_s0, sem_s1)
    sem_n = (sem_n0, sem_n1)
    sem_w = (sem_w0, sem_w1)
    pltpu.sync_copy(b_hbm, b_v)
    sq_v[0, :] = jnp.zeros((16,), jnp.float32)
    sq_v[1, :] = jnp.zeros((16,), jnp.float32)

    def issue(buf, r0):
        @pl.when(r0 < NB)
        def _():
            pltpu.sync_copy(nid_hbm.at[pl.ds(r0, CH)], idxs_v[buf])
            pltpu.sync_copy(n1f_hbm.at[pl.ds(r0 * NT, CH * NT)], idxn_v[buf])
            pltpu.sync_copy(a1f_hbm.at[pl.ds(r0 * NT, CH * NT)],
                            al_v[buf].at[pl.ds(0, CH * NT)])

        @pl.when(r0 >= NB)
        def _():
            e0 = r0 - NB
            pltpu.sync_copy(n1f_hbm.at[pl.ds(e0, CH)], idxs_v[buf])
            pltpu.sync_copy(n2f_hbm.at[pl.ds(e0 * NT, CH * NT)], idxn_v[buf])
            pltpu.sync_copy(a2f_hbm.at[pl.ds(e0 * NT, CH * NT)],
                            al_v[buf].at[pl.ds(0, CH * NT)])

        pltpu.async_copy(sp_hbm.at[idxs_v[buf]], self_v[buf], sem_s[buf])
        pltpu.async_copy(ap_hbm.at[idxn_v[buf]], nbr_v[buf], sem_n[buf])

    def wait(buf):
        pltpu.make_async_copy(sp_hbm.at[idxs_v[buf]], self_v[buf], sem_s[buf]).wait()
        pltpu.make_async_copy(ap_hbm.at[idxn_v[buf]], nbr_v[buf], sem_n[buf]).wait()

    def compute(buf, r0, first):
        @pl.when(jnp.logical_not(first))
        def _():
            pltpu.make_async_copy(out_v[buf], e1t_hbm.at[pl.ds(0, CH)],
                                  sem_w[buf]).wait()

        def row(i, sqc):
            al_vec = al_v[buf][pl.ds(i * NT, 16)]
            als = [al_vec[t] for t in range(NT)]
            for c in range(DD // 16):
                sl = pl.ds(c * 16, 16)
                acc = self_v[buf][i, sl] + b_v[sl]
                for t in range(NT):
                    acc = acc + als[t] * nbr_v[buf][i * NT + t, sl]
                r = jnp.maximum(acc, 0.0)
                out_v[buf][i, sl] = r
                sqc = sqc + r * r
            return sqc

        sqc = lax.fori_loop(0, CH, row, jnp.zeros((16,), jnp.float32))

        @pl.when(r0 < NB)
        def _():
            sq_v[0, :] = sq_v[0, :] + sqc
            pltpu.async_copy(out_v[buf], e1t_hbm.at[pl.ds(r0, CH)], sem_w[buf])

        @pl.when(r0 >= NB)
        def _():
            sq_v[1, :] = sq_v[1, :] + sqc
            pltpu.async_copy(out_v[buf], e1n_hbm.at[pl.ds(r0 - NB, CH)], sem_w[buf])

    issue(0, base)

    def pair(k, carry):
        r_a = base + (2 * k) * CH
        r_b = r_a + CH
        issue(1, r_b)
        wait(0)
        compute(0, r_a, k == 0)

        @pl.when(k + 1 < NPAIR)
        def _():
            issue(0, r_b + CH)

        wait(1)
        compute(1, r_b, k == 0)
        return carry

    lax.fori_loop(0, NPAIR, pair, 0)
    pltpu.make_async_copy(out_v[0], e1t_hbm.at[pl.ds(0, CH)], sem_w[0]).wait()
    pltpu.make_async_copy(out_v[1], e1t_hbm.at[pl.ds(0, CH)], sem_w[1]).wait()
    pltpu.sync_copy(sq_v.at[0], sqt_hbm.at[wid])
    pltpu.sync_copy(sq_v.at[1], sqn_hbm.at[wid])


_sc_call = functools.partial(
    pl.kernel,
    out_type=(
        jax.ShapeDtypeStruct((NB, DD), jnp.float32),
        jax.ShapeDtypeStruct((NB * NT, DD), jnp.float32),
        jax.ShapeDtypeStruct((NW, 16), jnp.float32),
        jax.ShapeDtypeStruct((NW, 16), jnp.float32),
    ),
    mesh=plsc.VectorSubcoreMesh(core_axis_name="c", subcore_axis_name="s"),
    scratch_types=[
        pltpu.VMEM((CH,), jnp.int32),
        pltpu.VMEM((CH,), jnp.int32),
        pltpu.VMEM((CH * NT,), jnp.int32),
        pltpu.VMEM((CH * NT,), jnp.int32),
        pltpu.VMEM((CH * NT + 16,), jnp.float32),
        pltpu.VMEM((CH * NT + 16,), jnp.float32),
        pltpu.VMEM((CH, DD), jnp.float32),
        pltpu.VMEM((CH, DD), jnp.float32),
        pltpu.VMEM((CH * NT, DD), jnp.float32),
        pltpu.VMEM((CH * NT, DD), jnp.float32),
        pltpu.VMEM((CH, DD), jnp.float32),
        pltpu.VMEM((CH, DD), jnp.float32),
        pltpu.VMEM((DD,), jnp.float32),
        pltpu.VMEM((2, 16), jnp.float32),
        pltpu.SemaphoreType.DMA,
        pltpu.SemaphoreType.DMA,
        pltpu.SemaphoreType.DMA,
        pltpu.SemaphoreType.DMA,
        pltpu.SemaphoreType.DMA,
        pltpu.SemaphoreType.DMA,
    ],
)(_sc_body)


_BM0 = 512


def _t3b_body(e1t_ref, e1n_ref, al_ref, ws0, bs0, wa0, ba0, wo0t, wo0b, bo0,
              sqt_ref, sqn_ref, e0_ref, sq_ref):
    nu1 = jnp.sqrt(jnp.sum(sqt_ref[...]))
    nu2 = jnp.sqrt(jnp.sum(sqn_ref[...]))
    zt = jnp.maximum(
        jnp.dot(e1t_ref[...], ws0[...], preferred_element_type=jnp.float32)
        + nu1 * bs0[...], 0.0)
    al = al_ref[...]
    agg = jnp.zeros((_BM0, DD), jnp.float32)
    for t in range(NT):
        znt = jnp.maximum(
            jnp.dot(e1n_ref[:, t, :], wa0[...], preferred_element_type=jnp.float32)
            + nu2 * ba0[...], 0.0)
        agg = agg + al[:, t:t + 1] * znt
    pre = (jnp.dot(zt, wo0t[...], preferred_element_type=jnp.float32) / nu1
           + jnp.dot(agg, wo0b[...], preferred_element_type=jnp.float32) / nu2
           + bo0[...])
    e0 = jnp.maximum(pre, 0.0)
    e0_ref[...] = e0

    @pl.when(pl.program_id(0) == 0)
    def _():
        sq_ref[...] = jnp.zeros_like(sq_ref)

    sq_ref[...] += jnp.sum(e0 * e0, axis=0, keepdims=True)


def _t3b(e1t, e1n3, al1, ws0, bs0, wa0, ba0, wo0t, wo0b, bo0, sqt, sqn):
    full = pl.BlockSpec((DD, DD), lambda i: (0, 0))
    row = pl.BlockSpec((1, DD), lambda i: (0, 0))
    sqspec = pl.BlockSpec((NW, 16), lambda i: (0, 0))
    return pl.pallas_call(
        _t3b_body,
        grid=(NB // _BM0,),
        in_specs=[
            pl.BlockSpec((_BM0, DD), lambda i: (i, 0)),
            pl.BlockSpec((_BM0, NT, DD), lambda i: (i, 0, 0)),
            pl.BlockSpec((_BM0, NT), lambda i: (i, 0)),
            full, row, full, row, full, full, row,
            sqspec, sqspec,
        ],
        out_specs=[
            pl.BlockSpec((_BM0, DD), lambda i: (i, 0)),
            pl.BlockSpec((1, DD), lambda i: (0, 0)),
        ],
        out_shape=[
            jax.ShapeDtypeStruct((NB, DD), jnp.float32),
            jax.ShapeDtypeStruct((1, DD), jnp.float32),
        ],
    )(e1t, e1n3, al1, ws0, bs0, wa0, ba0, wo0t, wo0b, bo0, sqt, sqn)


def _t3c_body(e0_ref, wemb, bemb, sq_ref, out_ref):
    nu0 = jnp.sqrt(jnp.sum(sq_ref[...]))
    q = jnp.maximum(
        jnp.dot(e0_ref[...], wemb[...], preferred_element_type=jnp.float32)
        + nu0 * bemb[...], 0.0)
    out_ref[...] = q * (1.0 / nu0)


def _t3c(e0, wemb, bemb, sq0):
    return pl.pallas_call(
        _t3c_body,
        grid=(2,),
        in_specs=[
            pl.BlockSpec((NB // 2, DD), lambda i: (i, 0)),
            pl.BlockSpec((DD, DD), lambda i: (0, 0)),
            pl.BlockSpec((1, DD), lambda i: (0, 0)),
            pl.BlockSpec((1, DD), lambda i: (0, 0)),
        ],
        out_specs=pl.BlockSpec((NB // 2, DD), lambda i: (i, 0)),
        out_shape=jax.ShapeDtypeStruct((NB, DD), jnp.float32),
    )(e0, wemb, bemb, sq0)


def kernel(x, node_ids, neigh1, alpha1, neigh2, alpha2,
           W_self1, b_self1, W_agg1, b_agg1, W_out1, b_out1,
           W_self0, b_self0, W_agg0, b_agg0, W_out0, b_out0,
           W_emb, b_emb):
    Sp, Ap = _t1(x, W_self1, b_self1.reshape(1, DD), W_agg1,
                 b_agg1.reshape(1, DD), W_out1[:DD], W_out1[DD:])

    e1_t, e1_n, sqt, sqn = _sc_call(Sp, Ap, node_ids, neigh1.reshape(-1),
                                    neigh2.reshape(-1), alpha1.reshape(-1),
                                    alpha2.reshape(-1), b_out1)

    e1n3 = e1_n.reshape(NB, NT, DD)
    e0, sq0 = _t3b(e1_t, e1n3, alpha1, W_self0, b_self0.reshape(1, DD),
                   W_agg0, b_agg0.reshape(1, DD), W_out0[:DD], W_out0[DD:],
                   b_out0.reshape(1, DD), sqt, sqn)

    return _t3c(e0, W_emb, b_emb.reshape(1, DD), sq0)
